# baseline (speedup 1.0000x reference)
import jax
import jax.numpy as jnp
from jax import lax
from jax.experimental import pallas as pl
from jax.experimental.pallas import tpu as pltpu
from jax.experimental.pallas import tpu_sc as plsc

FACTOR_REG = 0.0005
BIAS_REG = 0.01

B = 16384
K = 64
NQ = K // 16
NC = 2
NS = 16
NW = NC * NS
BPW = B // NW
HALF = BPW // 2
CHUNK = 128
VROWS = 1000000
PROWS = VROWS // 2

_SC_PARAMS = pltpu.CompilerParams(
    needs_layout_passes=False, use_tc_tiling_on_sc=True)
_MESH = plsc.VectorSubcoreMesh(core_axis_name="c", subcore_axis_name="s")


def _halved(dst, src):
    for t in range(BPW // 16):
        sl = pl.ds(t * 16, 16)
        dst[sl] = jax.lax.shift_right_logical(src[sl], 1)


def _select_row(rows, row, par_vec, r):
    s = jnp.full((16,), par_vec[r], jnp.int32) == 1
    return [jnp.where(s,
                      rows[row, pl.ds(64 + q * 16, 16)],
                      rows[row, pl.ds(q * 16, 16)])
            for q in range(NQ)]


def _u_body(u_hbm, ue2, xu_out, l2u_out,
            idx_u, hid_u, rows_u, xu_v, l2_v, sem):
    wid = lax.axis_index("c") * NS + lax.axis_index("s")
    base = wid * BPW

    pltpu.sync_copy(u_hbm.at[pl.ds(base, BPW)], idx_u)
    _halved(hid_u, idx_u)

    copies = []
    for c in range(BPW // CHUNK):
        sl = pl.ds(c * CHUNK, CHUNK)
        copies.append(pltpu.async_copy(ue2.at[hid_u.at[sl]], rows_u.at[sl], sem))
    for cp in copies:
        cp.wait()

    zf = jnp.zeros((16,), jnp.float32)

    def group(gg, l2f):
        rb = gg * 16
        par_u = jnp.bitwise_and(idx_u[pl.ds(rb, 16)], 1)
        for r in range(16):
            pu = _select_row(rows_u, rb + r, par_u, r)
            for q in range(NQ):
                l2f = l2f + pu[q] * pu[q]
                xu_v[pl.ds((rb + r) * K + q * 16, 16)] = pu[q]
        return l2f

    l2f = lax.fori_loop(0, BPW // 16, group, zf)
    l2_v[...] = l2f
    pltpu.sync_copy(xu_v, xu_out.at[pl.ds(base * K, BPW * K)])
    pltpu.sync_copy(l2_v, l2u_out.at[pl.ds(wid * 16, 16)])


_u_call = pl.kernel(
    _u_body,
    out_type=(
        jax.ShapeDtypeStruct((B * K,), jnp.float32),
        jax.ShapeDtypeStruct((NW * 16,), jnp.float32),
    ),
    mesh=_MESH,
    compiler_params=_SC_PARAMS,
    scratch_types=[
        pltpu.VMEM((BPW,), jnp.int32),
        pltpu.VMEM((BPW,), jnp.int32),
        pltpu.VMEM((BPW, 128), jnp.float32),
        pltpu.VMEM((BPW * K,), jnp.float32),
        pltpu.VMEM((16,), jnp.float32),
        pltpu.SemaphoreType.DMA,
    ],
)


def _i_body(i_hbm, j_hbm, ie2, ibf, xu_hbm,
            xuij_out, l2r_out,
            idx_i, idx_j, hid_i, hid_j, xu_v,
            rows_i, rows_j, ibv_buf, jbv_buf, xuij_v, l2_v, sem):
    wid = lax.axis_index("c") * NS + lax.axis_index("s")
    base = wid * BPW

    pltpu.sync_copy(i_hbm.at[pl.ds(base, BPW)], idx_i)
    pltpu.sync_copy(j_hbm.at[pl.ds(base, BPW)], idx_j)
    pltpu.sync_copy(xu_hbm.at[pl.ds(base * K, BPW * K)], xu_v)
    _halved(hid_i, idx_i)
    _halved(hid_j, idx_j)

    lane = lax.iota(jnp.int32, 16)
    zf = jnp.zeros((16,), jnp.float32)

    def gather_half(h):
        copies = []
        for c in range(HALF // CHUNK):
            src = pl.ds(h * HALF + c * CHUNK, CHUNK)
            dst = pl.ds(c * CHUNK, CHUNK)
            copies.append(pltpu.async_copy(
                ie2.at[hid_i.at[src]], rows_i.at[dst], sem))
            copies.append(pltpu.async_copy(
                ie2.at[hid_j.at[src]], rows_j.at[dst], sem))
            copies.append(pltpu.async_copy(
                ibf.at[idx_i.at[src]], ibv_buf.at[dst], sem))
            copies.append(pltpu.async_copy(
                ibf.at[idx_j.at[src]], jbv_buf.at[dst], sem))
        for cp in copies:
            cp.wait()

    def compute_half(h, carry):
        def group(gg, carry):
            l2f, l2ib, l2jb = carry
            rb = gg * 16
            par_i = jnp.bitwise_and(idx_i[pl.ds(h * HALF + rb, 16)], 1)
            par_j = jnp.bitwise_and(idx_j[pl.ds(h * HALF + rb, 16)], 1)
            xvec = zf
            for r in range(16):
                pi = _select_row(rows_i, rb + r, par_i, r)
                pj = _select_row(rows_j, rb + r, par_j, r)
                ubase = (h * HALF + rb + r) * K
                pu = [xu_v[pl.ds(ubase + q * 16, 16)] for q in range(NQ)]
                di = zf
                dj = zf
                for q in range(NQ):
                    di = di + pu[q] * pi[q]
                    dj = dj + pu[q] * pj[q]
                    l2f = l2f + pi[q] * pi[q]
                    l2f = l2f + pj[q] * pj[q]
                d = jnp.sum(di - dj)
                xvec = jnp.where(lane == r, d, xvec)
            ibv = ibv_buf[pl.ds(rb, 16)]
            jbv = jbv_buf[pl.ds(rb, 16)]
            xuij_v[pl.ds(h * HALF + rb, 16)] = xvec + (ibv - jbv)
            l2ib = l2ib + ibv * ibv
            l2jb = l2jb + jbv * jbv
            return l2f, l2ib, l2jb

        return lax.fori_loop(0, HALF // 16, group, carry)

    carry = (zf, zf, zf)
    gather_half(0)
    carry = compute_half(0, carry)
    gather_half(1)
    carry = compute_half(1, carry)
    l2f, l2ib, l2jb = carry

    l2_v[...] = (jnp.float32(FACTOR_REG) * l2f
                 + jnp.float32(BIAS_REG) * l2ib
                 + jnp.float32(BIAS_REG / 10.0) * l2jb)

    pltpu.sync_copy(xuij_v, xuij_out.at[pl.ds(base, BPW)])
    pltpu.sync_copy(l2_v, l2r_out.at[pl.ds(wid * 16, 16)])


_i_call = pl.kernel(
    _i_body,
    out_type=(
        jax.ShapeDtypeStruct((B,), jnp.float32),
        jax.ShapeDtypeStruct((NW * 16,), jnp.float32),
    ),
    mesh=_MESH,
    compiler_params=_SC_PARAMS,
    scratch_types=[
        pltpu.VMEM((BPW,), jnp.int32),
        pltpu.VMEM((BPW,), jnp.int32),
        pltpu.VMEM((BPW,), jnp.int32),
        pltpu.VMEM((BPW,), jnp.int32),
        pltpu.VMEM((BPW * K,), jnp.float32),
        pltpu.VMEM((HALF, 128), jnp.float32),
        pltpu.VMEM((HALF, 128), jnp.float32),
        pltpu.VMEM((HALF,), jnp.float32),
        pltpu.VMEM((HALF,), jnp.float32),
        pltpu.VMEM((BPW,), jnp.float32),
        pltpu.VMEM((16,), jnp.float32),
        pltpu.SemaphoreType.DMA,
    ],
)


def _tc_body(x_ref, l2u_ref, l2r_ref, loss_ref, auc_ref):
    x = x_ref[...]
    l2 = (jnp.float32(FACTOR_REG) * jnp.sum(l2u_ref[...])
          + jnp.sum(l2r_ref[...]))
    logsig = jnp.sum(jnp.log(jax.nn.sigmoid(x)))
    auc = jnp.sum((x > 0).astype(jnp.float32))
    loss_ref[0, 0] = l2 - logsig / jnp.float32(B)
    auc_ref[0, 0] = auc / jnp.float32(B)


_tc_call = pl.pallas_call(
    _tc_body,
    out_shape=(
        jax.ShapeDtypeStruct((1, 1), jnp.float32),
        jax.ShapeDtypeStruct((1, 1), jnp.float32),
    ),
    out_specs=(
        pl.BlockSpec(memory_space=pltpu.SMEM),
        pl.BlockSpec(memory_space=pltpu.SMEM),
    ),
)


def kernel(u, i, j, user_emb_w, item_emb_w, item_b):
    u32 = u.astype(jnp.int32)
    i32 = i.astype(jnp.int32)
    j32 = j.astype(jnp.int32)
    ue2 = user_emb_w[:VROWS].reshape(PROWS, 128)
    ie2 = item_emb_w[:VROWS].reshape(PROWS, 128)
    ib_flat = item_b.reshape(-1)
    xu, l2u = _u_call(u32, ue2)
    xuij, l2r = _i_call(i32, j32, ie2, ib_flat, xu)
    loss, auc = _tc_call(xuij.reshape(128, 128), l2u.reshape(4, 128),
                         l2r.reshape(4, 128))
    return (loss[0, 0], auc[0, 0])


# speedup vs baseline: 1.0427x; 1.0427x over previous
import jax
import jax.numpy as jnp
from jax import lax
from jax.experimental import pallas as pl
from jax.experimental.pallas import tpu as pltpu
from jax.experimental.pallas import tpu_sc as plsc

FACTOR_REG = 0.0005
BIAS_REG = 0.01

B = 16384
K = 64
NC = 2
NS = 16
NW = NC * NS
BPW = B // NW
CHUNK = 128
NCHUNK = BPW // CHUNK


def _sc_body(u_r, i_r, j_r, ue_hbm, ie_hbm, ib_hbm,
             xuij_out, l2_out,
             idx_u, idx_i, idx_j, rows_u, rows_i, rows_j,
             ibv_buf, jbv_buf, xuij_v, l2_v, sem):
    wid = lax.axis_index("c") * NS + lax.axis_index("s")

    pltpu.sync_copy(u_r.at[pl.ds(wid * NCHUNK, NCHUNK)], idx_u)
    pltpu.sync_copy(i_r.at[pl.ds(wid * NCHUNK, NCHUNK)], idx_i)
    pltpu.sync_copy(j_r.at[pl.ds(wid * NCHUNK, NCHUNK)], idx_j)

    copies = []
    for c in range(NCHUNK):
        sl = pl.ds(c * CHUNK, CHUNK)
        copies.append(pltpu.async_copy(ue_hbm.at[idx_u.at[c]], rows_u.at[sl], sem))
        copies.append(pltpu.async_copy(ie_hbm.at[idx_i.at[c]], rows_i.at[sl], sem))
        copies.append(pltpu.async_copy(ie_hbm.at[idx_j.at[c]], rows_j.at[sl], sem))
        copies.append(pltpu.async_copy(ib_hbm.at[idx_i.at[c]], ibv_buf.at[sl], sem))
        copies.append(pltpu.async_copy(ib_hbm.at[idx_j.at[c]], jbv_buf.at[sl], sem))
    for cp in copies:
        cp.wait()

    lane = lax.iota(jnp.int32, 16)
    zf = jnp.zeros((16,), jnp.float32)

    def group(gg, carry):
        l2f, l2ib, l2jb = carry
        rb = gg * 16
        xvec = zf
        for r in range(16):
            row = rb + r
            pu = [rows_u[row, pl.ds(q * 16, 16)] for q in range(K // 16)]
            pi = [rows_i[row, pl.ds(q * 16, 16)] for q in range(K // 16)]
            pj = [rows_j[row, pl.ds(q * 16, 16)] for q in range(K // 16)]
            di = zf
            dj = zf
            for q in range(K // 16):
                di = di + pu[q] * pi[q]
                dj = dj + pu[q] * pj[q]
                l2f = l2f + pu[q] * pu[q]
                l2f = l2f + pi[q] * pi[q]
                l2f = l2f + pj[q] * pj[q]
            d = jnp.sum(di - dj)
            xvec = jnp.where(lane == r, d, xvec)
        ibv = ibv_buf[pl.ds(rb, 16)]
        jbv = jbv_buf[pl.ds(rb, 16)]
        x = xvec + (ibv - jbv)
        xuij_v[pl.ds(rb, 16)] = x
        l2ib = l2ib + ibv * ibv
        l2jb = l2jb + jbv * jbv
        return l2f, l2ib, l2jb

    l2f, l2ib, l2jb = lax.fori_loop(0, BPW // 16, group, (zf, zf, zf))
    l2_v[...] = (jnp.float32(FACTOR_REG) * l2f
                 + jnp.float32(BIAS_REG) * l2ib
                 + jnp.float32(BIAS_REG / 10.0) * l2jb)

    pltpu.sync_copy(xuij_v, xuij_out.at[pl.ds(wid * BPW, BPW)])
    pltpu.sync_copy(l2_v, l2_out.at[wid])


_sc_call = pl.kernel(
    _sc_body,
    out_type=(
        jax.ShapeDtypeStruct((B,), jnp.float32),
        jax.ShapeDtypeStruct((NW, 16), jnp.float32),
    ),
    mesh=plsc.VectorSubcoreMesh(core_axis_name="c", subcore_axis_name="s"),
    compiler_params=pltpu.CompilerParams(
        needs_layout_passes=False, use_tc_tiling_on_sc=False),
    scratch_types=[
        pltpu.VMEM((NCHUNK, CHUNK), jnp.int32),
        pltpu.VMEM((NCHUNK, CHUNK), jnp.int32),
        pltpu.VMEM((NCHUNK, CHUNK), jnp.int32),
        pltpu.VMEM((BPW, K), jnp.float32),
        pltpu.VMEM((BPW, K), jnp.float32),
        pltpu.VMEM((BPW, K), jnp.float32),
        pltpu.VMEM((BPW,), jnp.float32),
        pltpu.VMEM((BPW,), jnp.float32),
        pltpu.VMEM((BPW,), jnp.float32),
        pltpu.VMEM((16,), jnp.float32),
        pltpu.SemaphoreType.DMA,
    ],
)


def _tc_body(x_ref, l2_ref, loss_ref, auc_ref):
    x = x_ref[...]
    l2 = jnp.sum(l2_ref[...])
    logsig = jnp.sum(jnp.log(jax.nn.sigmoid(x)))
    auc = jnp.sum((x > 0).astype(jnp.float32))
    loss_ref[0, 0] = l2 - logsig / jnp.float32(B)
    auc_ref[0, 0] = auc / jnp.float32(B)


_tc_call = pl.pallas_call(
    _tc_body,
    out_shape=(
        jax.ShapeDtypeStruct((1, 1), jnp.float32),
        jax.ShapeDtypeStruct((1, 1), jnp.float32),
    ),
    out_specs=(
        pl.BlockSpec(memory_space=pltpu.SMEM),
        pl.BlockSpec(memory_space=pltpu.SMEM),
    ),
)


def kernel(u, i, j, user_emb_w, item_emb_w, item_b):
    u_r = u.astype(jnp.int32).reshape(NW * NCHUNK, CHUNK)
    i_r = i.astype(jnp.int32).reshape(NW * NCHUNK, CHUNK)
    j_r = j.astype(jnp.int32).reshape(NW * NCHUNK, CHUNK)
    ib_flat = item_b.reshape(-1)
    xuij, l2p = _sc_call(u_r, i_r, j_r, user_emb_w, item_emb_w, ib_flat)
    loss, auc = _tc_call(xuij.reshape(128, 128), l2p)
    return (loss[0, 0], auc[0, 0])
